# Initial kernel scaffold; baseline (speedup 1.0000x reference)
import jax
import jax.numpy as jnp
from jax.experimental import pallas as pl


def kernel(ally_ids, enemy_ids, pos_hero_id, neg_hero_id, hero_emb, static_feats, sp_w, sp_b, sp_g, sp_bb, cp_w1, cp_b1, cp_g, cp_bb, cp_w2, cp_b2):
    raise NotImplementedError("write your pallas kernel here")


# speedup vs baseline: 7.8738x; 7.8738x over previous
import functools

import jax
import jax.numpy as jnp
from jax import lax
from jax.experimental import pallas as pl
from jax.experimental.pallas import tpu as pltpu
from jax.experimental.pallas import tpu_sc as plsc

EPS = 1e-5

NC = 2
NS = 16
NW = NC * NS
LANES = 16

D = 128
VP = 1024


def _tables_body(hero_ref, stat_ref, sp_w_ref, sp_b_ref, sp_g_ref,
                 sp_bb_ref, cp_w1_ref, rep_ref, pa_ref, pe_ref):
    s = jnp.dot(stat_ref[...], sp_w_ref[...],
                preferred_element_type=jnp.float32) + sp_b_ref[...]
    mu = jnp.mean(s, axis=-1, keepdims=True)
    var = jnp.mean((s - mu) ** 2, axis=-1, keepdims=True)
    s = (s - mu) / jnp.sqrt(var + EPS) * sp_g_ref[...] + sp_bb_ref[...]
    rep = hero_ref[...] + 0.1 * jnp.tanh(s)
    rep_ref[...] = rep
    pa_ref[...] = jnp.dot(rep, cp_w1_ref[0:D, :],
                          preferred_element_type=jnp.float32) * 0.25
    pe_ref[...] = jnp.dot(rep, cp_w1_ref[D:2 * D, :],
                          preferred_element_type=jnp.float32) * (0.8 / 5.0)


def _tables(hero_p, stat_p, sp_w, sp_b, sp_g, sp_bb, cp_w1):
    out = jax.ShapeDtypeStruct((VP, D), jnp.float32)
    return pl.pallas_call(
        _tables_body,
        out_shape=(out, out, out),
    )(hero_p, stat_p, sp_w, sp_b.reshape(1, D), sp_g.reshape(1, D),
      sp_bb.reshape(1, D), cp_w1)


CHUNK = 32


def _gather_body(rep_hbm, pa_hbm, pe_hbm, aidx_hbm, eidx_hbm, pidx_hbm,
                 nidx_hbm, acc_hbm, pos_hbm, neg_hbm,
                 aidx_v, eidx_v, pidx_v, nidx_v,
                 arows, erows, prows, nrows, accv, sem, rows_per_w):
    wid = lax.axis_index("s") * NC + lax.axis_index("c")
    nchunks = rows_per_w // CHUNK
    for chunk in range(nchunks):
        base = wid * rows_per_w + chunk * CHUNK
        pltpu.sync_copy(aidx_hbm.at[pl.ds(base * 4, 4 * CHUNK)], aidx_v)
        pltpu.sync_copy(eidx_hbm.at[pl.ds(base * 5, 5 * CHUNK)], eidx_v)
        pltpu.sync_copy(pidx_hbm.at[pl.ds(base, CHUNK)], pidx_v)
        pltpu.sync_copy(nidx_hbm.at[pl.ds(base, CHUNK)], nidx_v)
        d1 = pltpu.async_copy(pa_hbm.at[aidx_v], arows, sem)
        d2 = pltpu.async_copy(pe_hbm.at[eidx_v.at[pl.ds(0, 128)]],
                              erows.at[pl.ds(0, 128)], sem)
        d3 = pltpu.async_copy(pe_hbm.at[eidx_v.at[pl.ds(128, 32)]],
                              erows.at[pl.ds(128, 32)], sem)
        d4 = pltpu.async_copy(rep_hbm.at[pidx_v], prows, sem)
        d5 = pltpu.async_copy(rep_hbm.at[nidx_v], nrows, sem)
        d1.wait()
        d2.wait()
        d3.wait()
        d4.wait()
        d5.wait()

        def body(i, carry):
            for dd in range(D // LANES):
                sl = pl.ds(dd * LANES, LANES)
                v = (arows[4 * i, sl] + arows[4 * i + 1, sl]
                     + arows[4 * i + 2, sl] + arows[4 * i + 3, sl])
                v = (v + erows[5 * i, sl] + erows[5 * i + 1, sl]
                     + erows[5 * i + 2, sl] + erows[5 * i + 3, sl]
                     + erows[5 * i + 4, sl])
                accv[i, sl] = v
            return carry

        lax.fori_loop(0, CHUNK, body, 0)
        pltpu.sync_copy(accv, acc_hbm.at[pl.ds(base, CHUNK)])
        pltpu.sync_copy(prows, pos_hbm.at[pl.ds(base, CHUNK)])
        pltpu.sync_copy(nrows, neg_hbm.at[pl.ds(base, CHUNK)])


def _gather(rep, pa, pe, aidx, eidx, pidx, nidx, batch):
    rows_per_w = batch // NW
    assert rows_per_w % CHUNK == 0
    mesh = plsc.VectorSubcoreMesh(core_axis_name="c", subcore_axis_name="s",
                                  num_cores=NC, num_subcores=NS)
    out = jax.ShapeDtypeStruct((batch, D), jnp.float32)
    fn = pl.kernel(
        functools.partial(_gather_body, rows_per_w=rows_per_w),
        out_type=(out, out, out),
        mesh=mesh,
        scratch_types=[
            pltpu.VMEM((4 * CHUNK,), jnp.int32),
            pltpu.VMEM((5 * CHUNK,), jnp.int32),
            pltpu.VMEM((CHUNK,), jnp.int32),
            pltpu.VMEM((CHUNK,), jnp.int32),
            pltpu.VMEM((4 * CHUNK, D), jnp.float32),
            pltpu.VMEM((5 * CHUNK, D), jnp.float32),
            pltpu.VMEM((CHUNK, D), jnp.float32),
            pltpu.VMEM((CHUNK, D), jnp.float32),
            pltpu.VMEM((CHUNK, D), jnp.float32),
            pltpu.SemaphoreType.DMA,
        ],
    )
    return fn(rep, pa, pe, aidx, eidx, pidx, nidx)


def _head_body(acc_ref, pos_ref, neg_ref, cp_b1_ref, cp_g_ref, cp_bb_ref,
               cp_w2_ref, cp_b2_ref, ps_ref, ns_ref):
    x = acc_ref[...] + cp_b1_ref[...]
    mu = jnp.mean(x, axis=-1, keepdims=True)
    var = jnp.mean((x - mu) ** 2, axis=-1, keepdims=True)
    h = (x - mu) / jnp.sqrt(var + EPS) * cp_g_ref[...] + cp_bb_ref[...]
    h = jnp.maximum(h, 0.0)
    cv = jnp.dot(h, cp_w2_ref[...],
                 preferred_element_type=jnp.float32) + cp_b2_ref[...]
    ps_ref[...] = jnp.sum(cv * pos_ref[...], axis=-1, keepdims=True)
    ns_ref[...] = jnp.sum(cv * neg_ref[...], axis=-1, keepdims=True)


def _head(acc, posv, negv, cp_b1, cp_g, cp_bb, cp_w2, cp_b2, batch):
    blk = 2048
    grid = (batch // blk,)
    bspec = pl.BlockSpec((blk, D), lambda i: (i, 0))
    wspec = pl.BlockSpec((1, D), lambda i: (0, 0))
    w2spec = pl.BlockSpec((D, D), lambda i: (0, 0))
    sspec = pl.BlockSpec((blk, 1), lambda i: (i, 0))
    out = jax.ShapeDtypeStruct((batch, 1), jnp.float32)
    return pl.pallas_call(
        _head_body,
        grid=grid,
        in_specs=[bspec, bspec, bspec, wspec, wspec, wspec, w2spec, wspec],
        out_specs=(sspec, sspec),
        out_shape=(out, out),
    )(acc, posv, negv, cp_b1.reshape(1, D), cp_g.reshape(1, D),
      cp_bb.reshape(1, D), cp_w2, cp_b2.reshape(1, D))


def kernel(ally_ids, enemy_ids, pos_hero_id, neg_hero_id, hero_emb,
           static_feats, sp_w, sp_b, sp_g, sp_bb, cp_w1, cp_b1, cp_g,
           cp_bb, cp_w2, cp_b2):
    batch = ally_ids.shape[0]
    v = hero_emb.shape[0]
    hero_p = jnp.pad(hero_emb, ((0, VP - v), (0, 0)))
    stat_p = jnp.pad(static_feats, ((0, VP - v), (0, 0)))
    aidx = ally_ids.reshape(-1).astype(jnp.int32)
    eidx = enemy_ids.reshape(-1).astype(jnp.int32)
    pidx = pos_hero_id.astype(jnp.int32)
    nidx = neg_hero_id.astype(jnp.int32)

    rep, pa, pe = _tables(hero_p, stat_p, sp_w, sp_b, sp_g, sp_bb, cp_w1)
    acc, posv, negv = _gather(rep, pa, pe, aidx, eidx, pidx, nidx, batch)
    ps, ns = _head(acc, posv, negv, cp_b1, cp_g, cp_bb, cp_w2, cp_b2, batch)
    return ps.reshape(batch), ns.reshape(batch)


# speedup vs baseline: 11.4768x; 1.4576x over previous
import functools

import jax
import jax.numpy as jnp
from jax import lax
from jax.experimental import pallas as pl
from jax.experimental.pallas import tpu as pltpu
from jax.experimental.pallas import tpu_sc as plsc

EPS = 1e-5

NC = 2
NS = 16
NW = NC * NS
LANES = 16

D = 128
VP = 1024


def _tables_body(hero_ref, stat_ref, sp_w_ref, sp_b_ref, sp_g_ref,
                 sp_bb_ref, cp_w1_ref, t_ref):
    s = jnp.dot(stat_ref[...], sp_w_ref[...],
                preferred_element_type=jnp.float32) + sp_b_ref[...]
    mu = jnp.mean(s, axis=-1, keepdims=True)
    var = jnp.mean((s - mu) ** 2, axis=-1, keepdims=True)
    s = (s - mu) / jnp.sqrt(var + EPS) * sp_g_ref[...] + sp_bb_ref[...]
    rep = hero_ref[...] + 0.1 * jnp.tanh(s)
    t_ref[0:VP, :] = jnp.dot(rep, cp_w1_ref[0:D, :],
                             preferred_element_type=jnp.float32) * 0.25
    t_ref[VP:2 * VP, :] = jnp.dot(rep, cp_w1_ref[D:2 * D, :],
                                  preferred_element_type=jnp.float32) * (0.8 / 5.0)
    t_ref[2 * VP:3 * VP, :] = rep


def _tables(hero_p, stat_p, sp_w, sp_b, sp_g, sp_bb, cp_w1):
    out = jax.ShapeDtypeStruct((3 * VP, D), jnp.float32)
    return pl.pallas_call(
        _tables_body,
        out_shape=out,
    )(hero_p, stat_p, sp_w, sp_b.reshape(1, D), sp_g.reshape(1, D),
      sp_bb.reshape(1, D), cp_w1)


CHUNK = 32
IDXW = 11 * CHUNK


def _gather_body(t_hbm, idx_hbm, acc_hbm, pos_hbm, neg_hbm,
                 idxv, r0, r1, p0, p1, p2, n0, n1, n2, o0, o1, o2,
                 sg0, sg1, so0, so1, so2, rows_per_w):
    wid = lax.axis_index("s") * NC + lax.axis_index("c")
    nchunks = rows_per_w // CHUNK
    rbuf = (r0, r1)
    pbuf = (p0, p1, p2)
    nbuf = (n0, n1, n2)
    obuf = (o0, o1, o2)
    sg = (sg0, sg1)
    so = (so0, so1, so2)

    pltpu.sync_copy(idx_hbm.at[pl.ds(wid * nchunks * IDXW, nchunks * IDXW)],
                    idxv)

    def fire(c):
        s2, s3 = c % 2, c % 3
        bi = c * IDXW
        r, sem = rbuf[s2], sg[s2]
        return [
            pltpu.async_copy(t_hbm.at[idxv.at[pl.ds(bi, 128)]],
                             r.at[pl.ds(0, 128)], sem),
            pltpu.async_copy(t_hbm.at[idxv.at[pl.ds(bi + 128, 128)]],
                             r.at[pl.ds(128, 128)], sem),
            pltpu.async_copy(t_hbm.at[idxv.at[pl.ds(bi + 256, 32)]],
                             r.at[pl.ds(256, 32)], sem),
            pltpu.async_copy(t_hbm.at[idxv.at[pl.ds(bi + 288, 32)]],
                             pbuf[s3], sem),
            pltpu.async_copy(t_hbm.at[idxv.at[pl.ds(bi + 320, 32)]],
                             nbuf[s3], sem),
        ]

    gd = {0: fire(0), 1: fire(1)}
    od = {}
    for c in range(nchunks):
        s2, s3 = c % 2, c % 3
        for dsc in gd.pop(c):
            dsc.wait()
        r, ov = rbuf[s2], obuf[s3]

        def body(i, carry):
            for dd in range(D // LANES):
                sl = pl.ds(dd * LANES, LANES)
                v = (r[4 * i, sl] + r[4 * i + 1, sl]
                     + r[4 * i + 2, sl] + r[4 * i + 3, sl])
                v = (v + r[128 + 5 * i, sl] + r[128 + 5 * i + 1, sl]
                     + r[128 + 5 * i + 2, sl] + r[128 + 5 * i + 3, sl]
                     + r[128 + 5 * i + 4, sl])
                ov[i, sl] = v
            return carry

        lax.fori_loop(0, CHUNK, body, 0)
        base = wid * rows_per_w + c * CHUNK
        od[c] = [
            pltpu.async_copy(ov, acc_hbm.at[pl.ds(base, CHUNK)], so[s3]),
            pltpu.async_copy(pbuf[s3], pos_hbm.at[pl.ds(base, CHUNK)], so[s3]),
            pltpu.async_copy(nbuf[s3], neg_hbm.at[pl.ds(base, CHUNK)], so[s3]),
        ]
        if c + 2 < nchunks:
            if c - 1 in od:
                for dsc in od.pop(c - 1):
                    dsc.wait()
            gd[c + 2] = fire(c + 2)
    for c in sorted(od):
        for dsc in od[c]:
            dsc.wait()


def _gather(table, idx, batch):
    rows_per_w = batch // NW
    assert rows_per_w % CHUNK == 0
    mesh = plsc.VectorSubcoreMesh(core_axis_name="c", subcore_axis_name="s",
                                  num_cores=NC, num_subcores=NS)
    out = jax.ShapeDtypeStruct((batch, D), jnp.float32)
    nchunks = rows_per_w // CHUNK
    row = lambda n: pltpu.VMEM((n, D), jnp.float32)
    fn = pl.kernel(
        functools.partial(_gather_body, rows_per_w=rows_per_w),
        out_type=(out, out, out),
        mesh=mesh,
        scratch_types=[
            pltpu.VMEM((nchunks * IDXW,), jnp.int32),
            row(288), row(288),
            row(CHUNK), row(CHUNK), row(CHUNK),
            row(CHUNK), row(CHUNK), row(CHUNK),
            row(CHUNK), row(CHUNK), row(CHUNK),
            pltpu.SemaphoreType.DMA, pltpu.SemaphoreType.DMA,
            pltpu.SemaphoreType.DMA, pltpu.SemaphoreType.DMA,
            pltpu.SemaphoreType.DMA,
        ],
    )
    return fn(table, idx)


def _head_body(acc_ref, pos_ref, neg_ref, cp_b1_ref, cp_g_ref, cp_bb_ref,
               cp_w2_ref, cp_b2_ref, ps_ref, ns_ref):
    x = acc_ref[...] + cp_b1_ref[...]
    mu = jnp.mean(x, axis=-1, keepdims=True)
    var = jnp.mean((x - mu) ** 2, axis=-1, keepdims=True)
    h = (x - mu) / jnp.sqrt(var + EPS) * cp_g_ref[...] + cp_bb_ref[...]
    h = jnp.maximum(h, 0.0)
    cv = jnp.dot(h, cp_w2_ref[...],
                 preferred_element_type=jnp.float32) + cp_b2_ref[...]
    ps_ref[...] = jnp.sum(cv * pos_ref[...], axis=-1, keepdims=True)
    ns_ref[...] = jnp.sum(cv * neg_ref[...], axis=-1, keepdims=True)


def _head(acc, posv, negv, cp_b1, cp_g, cp_bb, cp_w2, cp_b2, batch):
    blk = 2048
    grid = (batch // blk,)
    bspec = pl.BlockSpec((blk, D), lambda i: (i, 0))
    wspec = pl.BlockSpec((1, D), lambda i: (0, 0))
    w2spec = pl.BlockSpec((D, D), lambda i: (0, 0))
    sspec = pl.BlockSpec((blk, 1), lambda i: (i, 0))
    out = jax.ShapeDtypeStruct((batch, 1), jnp.float32)
    return pl.pallas_call(
        _head_body,
        grid=grid,
        in_specs=[bspec, bspec, bspec, wspec, wspec, wspec, w2spec, wspec],
        out_specs=(sspec, sspec),
        out_shape=(out, out),
    )(acc, posv, negv, cp_b1.reshape(1, D), cp_g.reshape(1, D),
      cp_bb.reshape(1, D), cp_w2, cp_b2.reshape(1, D))


def kernel(ally_ids, enemy_ids, pos_hero_id, neg_hero_id, hero_emb,
           static_feats, sp_w, sp_b, sp_g, sp_bb, cp_w1, cp_b1, cp_g,
           cp_bb, cp_w2, cp_b2):
    batch = ally_ids.shape[0]
    v = hero_emb.shape[0]
    hero_p = jnp.pad(hero_emb, ((0, VP - v), (0, 0)))
    stat_p = jnp.pad(static_feats, ((0, VP - v), (0, 0)))
    nch = batch // CHUNK
    a = ally_ids.astype(jnp.int32).reshape(nch, 4 * CHUNK)
    e = (enemy_ids.astype(jnp.int32) + VP).reshape(nch, 5 * CHUNK)
    p = (pos_hero_id.astype(jnp.int32) + 2 * VP).reshape(nch, CHUNK)
    n = (neg_hero_id.astype(jnp.int32) + 2 * VP).reshape(nch, CHUNK)
    idx = jnp.concatenate([a, e, p, n], axis=1).reshape(-1)

    table = _tables(hero_p, stat_p, sp_w, sp_b, sp_g, sp_bb, cp_w1)
    acc, posv, negv = _gather(table, idx, batch)
    ps, ns = _head(acc, posv, negv, cp_b1, cp_g, cp_bb, cp_w2, cp_b2, batch)
    return ps.reshape(batch), ns.reshape(batch)


# speedup vs baseline: 12.8039x; 1.1156x over previous
import functools

import jax
import jax.numpy as jnp
from jax import lax
from jax.experimental import pallas as pl
from jax.experimental.pallas import tpu as pltpu
from jax.experimental.pallas import tpu_sc as plsc

EPS = 1e-5

NC = 2
NS = 16
NW = NC * NS
LANES = 16

D = 128


def _tables_body(hero_ref, stat_ref, sp_w_ref, sp_b_ref, sp_g_ref,
                 sp_bb_ref, cp_w1_ref, pa_ref, pe_ref, rep_ref):
    s = jnp.dot(stat_ref[...], sp_w_ref[...],
                preferred_element_type=jnp.float32) + sp_b_ref[...]
    mu = jnp.mean(s, axis=-1, keepdims=True)
    var = jnp.mean((s - mu) ** 2, axis=-1, keepdims=True)
    s = (s - mu) / jnp.sqrt(var + EPS) * sp_g_ref[...] + sp_bb_ref[...]
    rep = hero_ref[...] + 0.1 * jnp.tanh(s)
    pa_ref[...] = jnp.dot(rep, cp_w1_ref[0:D, :],
                          preferred_element_type=jnp.float32) * 0.25
    pe_ref[...] = jnp.dot(rep, cp_w1_ref[D:2 * D, :],
                          preferred_element_type=jnp.float32) * (0.8 / 5.0)
    rep_ref[...] = rep


def _tables(hero_emb, static_feats, sp_w, sp_b, sp_g, sp_bb, cp_w1):
    v = hero_emb.shape[0]
    out = jax.ShapeDtypeStruct((v, D), jnp.float32)
    return pl.pallas_call(
        _tables_body,
        out_shape=(out, out, out),
    )(hero_emb, static_feats, sp_w, sp_b.reshape(1, D), sp_g.reshape(1, D),
      sp_bb.reshape(1, D), cp_w1)


CHUNK = 32


def _gather_body(pa_hbm, pe_hbm, rep_hbm, aidx_hbm, eidx_hbm, pidx_hbm,
                 nidx_hbm, acc_hbm, pos_hbm, neg_hbm,
                 aidv, eidv, pidv, nidv,
                 r0, r1, p0, p1, p2, n0, n1, n2, o0, o1, o2,
                 sg0, sg1, so0, so1, so2, rows_per_w):
    wid = lax.axis_index("s") * NC + lax.axis_index("c")
    nchunks = rows_per_w // CHUNK
    rbuf = (r0, r1)
    pbuf = (p0, p1, p2)
    nbuf = (n0, n1, n2)
    obuf = (o0, o1, o2)
    sg = (sg0, sg1)
    so = (so0, so1, so2)
    base0 = wid * rows_per_w

    pltpu.sync_copy(aidx_hbm.at[pl.ds(base0 * 4, 4 * rows_per_w)], aidv)
    pltpu.sync_copy(eidx_hbm.at[pl.ds(base0 * 5, 5 * rows_per_w)], eidv)
    pltpu.sync_copy(pidx_hbm.at[pl.ds(base0, rows_per_w)], pidv)
    pltpu.sync_copy(nidx_hbm.at[pl.ds(base0, rows_per_w)], nidv)

    def fire(c):
        s2, s3 = c % 2, c % 3
        r, sem = rbuf[s2], sg[s2]
        return [
            pltpu.async_copy(pa_hbm.at[aidv.at[pl.ds(c * 128, 128)]],
                             r.at[pl.ds(0, 128)], sem),
            pltpu.async_copy(pe_hbm.at[eidv.at[pl.ds(c * 160, 128)]],
                             r.at[pl.ds(128, 128)], sem),
            pltpu.async_copy(pe_hbm.at[eidv.at[pl.ds(c * 160 + 128, 32)]],
                             r.at[pl.ds(256, 32)], sem),
            pltpu.async_copy(rep_hbm.at[pidv.at[pl.ds(c * CHUNK, CHUNK)]],
                             pbuf[s3], sem),
            pltpu.async_copy(rep_hbm.at[nidv.at[pl.ds(c * CHUNK, CHUNK)]],
                             nbuf[s3], sem),
        ]

    gd = {0: fire(0), 1: fire(1)}
    od = {}
    for c in range(nchunks):
        s2, s3 = c % 2, c % 3
        for dsc in gd.pop(c):
            dsc.wait()
        r, ov = rbuf[s2], obuf[s3]

        def body(i, carry):
            rows = [4 * i, 4 * i + 1, 4 * i + 2, 4 * i + 3,
                    128 + 5 * i, 128 + 5 * i + 1, 128 + 5 * i + 2,
                    128 + 5 * i + 3, 128 + 5 * i + 4]
            for dd in range(D // LANES):
                sl = pl.ds(dd * LANES, LANES)
                v = None
                for rr in rows:
                    x = r[rr, sl]
                    v = x if v is None else v + x
                ov[i, sl] = v
            return carry

        lax.fori_loop(0, CHUNK, body, 0)
        base = base0 + c * CHUNK
        od[c] = [
            pltpu.async_copy(ov, acc_hbm.at[pl.ds(base, CHUNK)], so[s3]),
            pltpu.async_copy(pbuf[s3], pos_hbm.at[pl.ds(base, CHUNK)], so[s3]),
            pltpu.async_copy(nbuf[s3], neg_hbm.at[pl.ds(base, CHUNK)], so[s3]),
        ]
        if c + 2 < nchunks:
            if c - 1 in od:
                for dsc in od.pop(c - 1):
                    dsc.wait()
            gd[c + 2] = fire(c + 2)
    for c in sorted(od):
        for dsc in od[c]:
            dsc.wait()


def _gather(pa, pe, rep, aidx, eidx, pidx, nidx, batch):
    rows_per_w = batch // NW
    assert rows_per_w % CHUNK == 0
    mesh = plsc.VectorSubcoreMesh(core_axis_name="c", subcore_axis_name="s",
                                  num_cores=NC, num_subcores=NS)
    out_t = jax.ShapeDtypeStruct((batch, D), jnp.float32)
    row = lambda n: pltpu.VMEM((n, D), jnp.float32)
    fn = pl.kernel(
        functools.partial(_gather_body, rows_per_w=rows_per_w),
        out_type=(out_t, out_t, out_t),
        mesh=mesh,
        scratch_types=[
            pltpu.VMEM((4 * rows_per_w,), jnp.int32),
            pltpu.VMEM((5 * rows_per_w,), jnp.int32),
            pltpu.VMEM((rows_per_w,), jnp.int32),
            pltpu.VMEM((rows_per_w,), jnp.int32),
            row(288), row(288),
            row(CHUNK), row(CHUNK), row(CHUNK),
            row(CHUNK), row(CHUNK), row(CHUNK),
            row(CHUNK), row(CHUNK), row(CHUNK),
            pltpu.SemaphoreType.DMA, pltpu.SemaphoreType.DMA,
            pltpu.SemaphoreType.DMA, pltpu.SemaphoreType.DMA,
            pltpu.SemaphoreType.DMA,
        ],
    )
    return fn(pa, pe, rep, aidx, eidx, pidx, nidx)


def _head_body(acc_ref, pos_ref, neg_ref, cp_b1_ref, cp_g_ref, cp_bb_ref,
               cp_w2_ref, cp_b2_ref, ps_ref, ns_ref, *, blk):
    x = acc_ref[...] + cp_b1_ref[...]
    mu = jnp.mean(x, axis=-1, keepdims=True)
    var = jnp.mean((x - mu) ** 2, axis=-1, keepdims=True)
    h = (x - mu) / jnp.sqrt(var + EPS) * cp_g_ref[...] + cp_bb_ref[...]
    h = jnp.maximum(h, 0.0)
    cv = jnp.dot(h, cp_w2_ref[...],
                 preferred_element_type=jnp.float32) + cp_b2_ref[...]
    ps = jnp.sum(cv * pos_ref[...], axis=-1)
    ns = jnp.sum(cv * neg_ref[...], axis=-1)
    ps_ref[...] = ps.reshape(blk // D, D)
    ns_ref[...] = ns.reshape(blk // D, D)


def _head(acc, posv, negv, cp_b1, cp_g, cp_bb, cp_w2, cp_b2, batch):
    blk = 2048
    grid = (batch // blk,)
    bspec = pl.BlockSpec((blk, D), lambda i: (i, 0))
    wspec = pl.BlockSpec((1, D), lambda i: (0, 0))
    w2spec = pl.BlockSpec((D, D), lambda i: (0, 0))
    sspec = pl.BlockSpec((blk // D, D), lambda i: (i, 0))
    out = jax.ShapeDtypeStruct((batch // D, D), jnp.float32)
    return pl.pallas_call(
        functools.partial(_head_body, blk=blk),
        grid=grid,
        in_specs=[bspec, bspec, bspec, wspec, wspec, wspec, w2spec, wspec],
        out_specs=(sspec, sspec),
        out_shape=(out, out),
    )(acc, posv, negv, cp_b1.reshape(1, D), cp_g.reshape(1, D),
      cp_bb.reshape(1, D), cp_w2, cp_b2.reshape(1, D))


def kernel(ally_ids, enemy_ids, pos_hero_id, neg_hero_id, hero_emb,
           static_feats, sp_w, sp_b, sp_g, sp_bb, cp_w1, cp_b1, cp_g,
           cp_bb, cp_w2, cp_b2):
    batch = ally_ids.shape[0]
    aidx = ally_ids.astype(jnp.int32).reshape(-1)
    eidx = enemy_ids.astype(jnp.int32).reshape(-1)
    pidx = pos_hero_id.astype(jnp.int32)
    nidx = neg_hero_id.astype(jnp.int32)

    pa, pe, rep = _tables(hero_emb, static_feats, sp_w, sp_b, sp_g, sp_bb,
                          cp_w1)
    acc, posv, negv = _gather(pa, pe, rep, aidx, eidx, pidx, nidx, batch)
    ps, ns = _head(acc, posv, negv, cp_b1, cp_g, cp_bb, cp_w2, cp_b2, batch)
    return ps.reshape(batch), ns.reshape(batch)


# speedup vs baseline: 16.0598x; 1.2543x over previous
import functools

import jax
import jax.numpy as jnp
from jax import lax
from jax.experimental import pallas as pl
from jax.experimental.pallas import tpu as pltpu
from jax.experimental.pallas import tpu_sc as plsc

EPS = 1e-5

NC = 2
NS = 16
NW = NC * NS
LANES = 16

D = 128


def _tables_body(hero_ref, stat_ref, sp_w_ref, sp_b_ref, sp_g_ref,
                 sp_bb_ref, cp_w1_ref, pa_ref, pe_ref, rep_ref):
    s = jnp.dot(stat_ref[...], sp_w_ref[...],
                preferred_element_type=jnp.float32) + sp_b_ref[...]
    mu = jnp.mean(s, axis=-1, keepdims=True)
    var = jnp.mean((s - mu) ** 2, axis=-1, keepdims=True)
    s = (s - mu) / jnp.sqrt(var + EPS) * sp_g_ref[...] + sp_bb_ref[...]
    rep = hero_ref[...] + 0.1 * jnp.tanh(s)
    pa_ref[...] = jnp.dot(rep, cp_w1_ref[0:D, :],
                          preferred_element_type=jnp.float32) * 0.25
    pe_ref[...] = jnp.dot(rep, cp_w1_ref[D:2 * D, :],
                          preferred_element_type=jnp.float32) * (0.8 / 5.0)
    rep_ref[...] = rep


def _tables(hero_emb, static_feats, sp_w, sp_b, sp_g, sp_bb, cp_w1):
    v = hero_emb.shape[0]
    out = jax.ShapeDtypeStruct((v, D), jnp.float32)
    return pl.pallas_call(
        _tables_body,
        out_shape=(out, out, out),
    )(hero_emb, static_feats, sp_w, sp_b.reshape(1, D), sp_g.reshape(1, D),
      sp_bb.reshape(1, D), cp_w1)


CHUNK = 32


def _gather_body(pa_hbm, pe_hbm, rep_hbm, a0h, a1h, a2h, a3h,
                 e0h, e1h, e2h, e3h, e4h, pidx_hbm,
                 nidx_hbm, acc_hbm, pos_hbm, neg_hbm,
                 av0, av1, av2, av3, ev0, ev1, ev2, ev3, ev4, pidv, nidv,
                 r0, r1, p0, p1, p2, n0, n1, n2, o0, o1, o2,
                 sg0, sg1, so0, so1, so2, rows_per_w):
    wid = lax.axis_index("s") * NC + lax.axis_index("c")
    nchunks = rows_per_w // CHUNK
    rbuf = (r0, r1)
    pbuf = (p0, p1, p2)
    nbuf = (n0, n1, n2)
    obuf = (o0, o1, o2)
    sg = (sg0, sg1)
    so = (so0, so1, so2)
    base0 = wid * rows_per_w
    av = (av0, av1, av2, av3)
    ev = (ev0, ev1, ev2, ev3, ev4)

    for hbm, buf in zip((a0h, a1h, a2h, a3h, e0h, e1h, e2h, e3h, e4h,
                         pidx_hbm, nidx_hbm),
                        av + ev + (pidv, nidv)):
        pltpu.sync_copy(hbm.at[pl.ds(base0, rows_per_w)], buf)

    def fire(c):
        s2, s3 = c % 2, c % 3
        r, sem = rbuf[s2], sg[s2]
        sl = pl.ds(c * CHUNK, CHUNK)
        ds = [pltpu.async_copy(pa_hbm.at[av[j].at[sl]],
                               r.at[pl.ds(32 * j, CHUNK)], sem)
              for j in range(4)]
        ds += [pltpu.async_copy(pe_hbm.at[ev[j].at[sl]],
                                r.at[pl.ds(128 + 32 * j, CHUNK)], sem)
               for j in range(5)]
        ds.append(pltpu.async_copy(rep_hbm.at[pidv.at[sl]], pbuf[s3], sem))
        ds.append(pltpu.async_copy(rep_hbm.at[nidv.at[sl]], nbuf[s3], sem))
        return ds

    gd = {0: fire(0), 1: fire(1)}
    od = {}
    for c in range(nchunks):
        s2, s3 = c % 2, c % 3
        for dsc in gd.pop(c):
            dsc.wait()
        r, ov = rbuf[s2], obuf[s3]

        def body(i, carry):
            rows = [i, 32 + i, 64 + i, 96 + i,
                    128 + i, 160 + i, 192 + i, 224 + i, 256 + i]
            for dd in range(D // LANES):
                sl = pl.ds(dd * LANES, LANES)
                v = None
                for rr in rows:
                    x = r[rr, sl]
                    v = x if v is None else v + x
                ov[i, sl] = v
            return carry

        lax.fori_loop(0, CHUNK, body, 0)
        base = base0 + c * CHUNK
        od[c] = [
            pltpu.async_copy(ov, acc_hbm.at[pl.ds(base, CHUNK)], so[s3]),
            pltpu.async_copy(pbuf[s3], pos_hbm.at[pl.ds(base, CHUNK)], so[s3]),
            pltpu.async_copy(nbuf[s3], neg_hbm.at[pl.ds(base, CHUNK)], so[s3]),
        ]
        if c + 2 < nchunks:
            if c - 1 in od:
                for dsc in od.pop(c - 1):
                    dsc.wait()
            gd[c + 2] = fire(c + 2)
    for c in sorted(od):
        for dsc in od[c]:
            dsc.wait()


def _gather(pa, pe, rep, aids, eids, pidx, nidx, batch):
    rows_per_w = batch // NW
    assert rows_per_w % CHUNK == 0
    mesh = plsc.VectorSubcoreMesh(core_axis_name="c", subcore_axis_name="s",
                                  num_cores=NC, num_subcores=NS)
    out_t = jax.ShapeDtypeStruct((batch, D), jnp.float32)
    row = lambda n: pltpu.VMEM((n, D), jnp.float32)
    idv = pltpu.VMEM((rows_per_w,), jnp.int32)
    fn = pl.kernel(
        functools.partial(_gather_body, rows_per_w=rows_per_w),
        out_type=(out_t, out_t, out_t),
        mesh=mesh,
        scratch_types=[
            idv, idv, idv, idv, idv, idv, idv, idv, idv, idv, idv,
            row(288), row(288),
            row(CHUNK), row(CHUNK), row(CHUNK),
            row(CHUNK), row(CHUNK), row(CHUNK),
            row(CHUNK), row(CHUNK), row(CHUNK),
            pltpu.SemaphoreType.DMA, pltpu.SemaphoreType.DMA,
            pltpu.SemaphoreType.DMA, pltpu.SemaphoreType.DMA,
            pltpu.SemaphoreType.DMA,
        ],
    )
    return fn(pa, pe, rep, *aids, *eids, pidx, nidx)


def _head_body(acc_ref, pos_ref, neg_ref, cp_b1_ref, cp_g_ref, cp_bb_ref,
               cp_w2_ref, cp_b2_ref, ps_ref, ns_ref, *, blk):
    x = acc_ref[...] + cp_b1_ref[...]
    mu = jnp.mean(x, axis=-1, keepdims=True)
    var = jnp.mean((x - mu) ** 2, axis=-1, keepdims=True)
    h = (x - mu) / jnp.sqrt(var + EPS) * cp_g_ref[...] + cp_bb_ref[...]
    h = jnp.maximum(h, 0.0)
    cv = jnp.dot(h, cp_w2_ref[...],
                 preferred_element_type=jnp.float32) + cp_b2_ref[...]
    ps = jnp.sum(cv * pos_ref[...], axis=-1)
    ns = jnp.sum(cv * neg_ref[...], axis=-1)
    ps_ref[...] = ps.reshape(blk // D, D)
    ns_ref[...] = ns.reshape(blk // D, D)


def _head(acc, posv, negv, cp_b1, cp_g, cp_bb, cp_w2, cp_b2, batch):
    blk = 2048
    grid = (batch // blk,)
    bspec = pl.BlockSpec((blk, D), lambda i: (i, 0))
    wspec = pl.BlockSpec((1, D), lambda i: (0, 0))
    w2spec = pl.BlockSpec((D, D), lambda i: (0, 0))
    sspec = pl.BlockSpec((blk // D, D), lambda i: (i, 0))
    out = jax.ShapeDtypeStruct((batch // D, D), jnp.float32)
    return pl.pallas_call(
        functools.partial(_head_body, blk=blk),
        grid=grid,
        in_specs=[bspec, bspec, bspec, wspec, wspec, wspec, w2spec, wspec],
        out_specs=(sspec, sspec),
        out_shape=(out, out),
    )(acc, posv, negv, cp_b1.reshape(1, D), cp_g.reshape(1, D),
      cp_bb.reshape(1, D), cp_w2, cp_b2.reshape(1, D))


def kernel(ally_ids, enemy_ids, pos_hero_id, neg_hero_id, hero_emb,
           static_feats, sp_w, sp_b, sp_g, sp_bb, cp_w1, cp_b1, cp_g,
           cp_bb, cp_w2, cp_b2):
    batch = ally_ids.shape[0]
    ally32 = ally_ids.astype(jnp.int32)
    enemy32 = enemy_ids.astype(jnp.int32)
    aids = [ally32[:, j] for j in range(4)]
    eids = [enemy32[:, j] for j in range(5)]
    pidx = pos_hero_id.astype(jnp.int32)
    nidx = neg_hero_id.astype(jnp.int32)

    pa, pe, rep = _tables(hero_emb, static_feats, sp_w, sp_b, sp_g, sp_bb,
                          cp_w1)
    acc, posv, negv = _gather(pa, pe, rep, aids, eids, pidx, nidx, batch)
    ps, ns = _head(acc, posv, negv, cp_b1, cp_g, cp_bb, cp_w2, cp_b2, batch)
    return ps.reshape(batch), ns.reshape(batch)


# speedup vs baseline: 16.6486x; 1.0367x over previous
import functools

import jax
import jax.numpy as jnp
import numpy as np
from jax import lax
from jax.experimental import pallas as pl
from jax.experimental.pallas import tpu as pltpu
from jax.experimental.pallas import tpu_sc as plsc

EPS = 1e-5

NC = 2
NS = 16
NW = NC * NS
LANES = 16

D = 128
DW = D // 2
VP = 1002

_HI = -65536

_PERM = np.empty(D, np.int32)
for _p in range(D):
    _k, _l = divmod(_p, 16)
    if _k % 2 == 0:
        _PERM[_p] = 8 * _k + _l
    else:
        _PERM[_p] = 64 + 8 * (_k - 1) + _l


def _pack_rows(x):
    rb = x.astype(jnp.bfloat16).astype(jnp.float32)
    lo = lax.shift_right_logical(
        lax.bitcast_convert_type(rb[:, 0:DW], jnp.int32), 16)
    hi = lax.bitcast_convert_type(rb[:, DW:D], jnp.int32) & _HI
    return lo | hi


def _tables_body(hero_ref, stat_ref, sp_w_ref, sp_b_ref, sp_g_ref,
                 sp_bb_ref, cp_w1_ref, pa_ref, pe_ref, rep_ref):
    s = jnp.dot(stat_ref[...], sp_w_ref[...],
                preferred_element_type=jnp.float32) + sp_b_ref[...]
    mu = jnp.mean(s, axis=-1, keepdims=True)
    var = jnp.mean((s - mu) ** 2, axis=-1, keepdims=True)
    s = (s - mu) / jnp.sqrt(var + EPS) * sp_g_ref[...] + sp_bb_ref[...]
    rep = hero_ref[...] + 0.1 * jnp.tanh(s)
    pa = jnp.dot(rep, cp_w1_ref[0:D, :],
                 preferred_element_type=jnp.float32) * 0.25
    pe = jnp.dot(rep, cp_w1_ref[D:2 * D, :],
                 preferred_element_type=jnp.float32) * (0.8 / 5.0)
    pa_ref[...] = _pack_rows(pa)
    pe_ref[...] = _pack_rows(pe)
    rep_ref[...] = rep


def _tables(hero_emb, static_feats, sp_w, sp_b, sp_g, sp_bb, cp_w1):
    v = hero_emb.shape[0]
    packed = jax.ShapeDtypeStruct((v, DW), jnp.int32)
    repf = jax.ShapeDtypeStruct((v, D), jnp.float32)
    return pl.pallas_call(
        _tables_body,
        out_shape=(packed, packed, repf),
    )(hero_emb, static_feats, sp_w, sp_b.reshape(1, D), sp_g.reshape(1, D),
      sp_bb.reshape(1, D), cp_w1)


CHUNK = 32


def _acc_loop(tab, idb, nids, ov_pair, acc_hbm, abase, nchunks, sem_o,
              pn_work, pn_finish):
    iot = [lax.iota(jnp.int32, 16) + 16 * g for g in range(4)]
    od = {}
    for c in range(nchunks):
        ov = ov_pair[c % 2]
        if c - 2 in od:
            od.pop(c - 2).wait()

        def body(i, carry):
            acc_lo = [None] * 4
            acc_hi = [None] * 4
            for j in range(nids):
                idj = idb[j][pl.ds(c * CHUNK + i, 16)][0]
                row = jnp.full((16,), lax.shift_right_logical(idj, 1),
                               dtype=jnp.int32)
                rem = jnp.full((16,), (idj & 1) * DW, dtype=jnp.int32)
                for g in range(4):
                    w = plsc.load_gather(tab, [row, rem + iot[g]])
                    hf = plsc.bitcast(w & _HI, jnp.float32)
                    lf = plsc.bitcast(w << 16, jnp.float32)
                    acc_lo[g] = lf if acc_lo[g] is None else acc_lo[g] + lf
                    acc_hi[g] = hf if acc_hi[g] is None else acc_hi[g] + hf
            for g in range(4):
                ov[i, pl.ds(32 * g, LANES)] = acc_lo[g]
                ov[i, pl.ds(32 * g + LANES, LANES)] = acc_hi[g]
            return carry

        lax.fori_loop(0, CHUNK, body, 0)
        od[c] = pltpu.async_copy(
            ov, acc_hbm.at[pl.ds(abase + c * CHUNK, CHUNK)], sem_o)
        pn_work(c)
    pn_finish()
    for c in sorted(od):
        od[c].wait()


def _gather_body(pa_hbm, pe_hbm, rep_hbm, a0h, a1h, a2h, a3h,
                 e0h, e1h, e2h, e3h, e4h, pidx_hbm, nidx_hbm,
                 acca_hbm, acce_hbm, pos_hbm, neg_hbm,
                 tab, i0, i1, i2, i3, i4, pidv, nidv,
                 pb0, pb1, nb0, nb1, ov0, ov1,
                 sg, so, sp, rows_acc, rows_pn):
    wid = lax.axis_index("s") * NC + lax.axis_index("c")
    is_a = wid < 16
    widr = lax.rem(wid, 16)
    abase = widr * rows_acc
    pnbase = wid * rows_pn
    nchunks = rows_acc // CHUNK
    pn_chunks = rows_pn // CHUNK
    idb = (i0, i1, i2, i3, i4)
    pbuf = (pb0, pb1)
    nbuf = (nb0, nb1)

    pltpu.sync_copy(pidx_hbm.at[pl.ds(pnbase, rows_pn)], pidv)
    pltpu.sync_copy(nidx_hbm.at[pl.ds(pnbase, rows_pn)], nidv)

    @pl.when(is_a)
    def _():
        pltpu.sync_copy(pa_hbm, tab)
        for j, h in enumerate((a0h, a1h, a2h, a3h)):
            pltpu.sync_copy(h.at[pl.ds(abase, rows_acc)],
                            idb[j].at[pl.ds(0, rows_acc)])

    @pl.when(jnp.logical_not(is_a))
    def _():
        pltpu.sync_copy(pe_hbm, tab)
        for j, h in enumerate((e0h, e1h, e2h, e3h, e4h)):
            pltpu.sync_copy(h.at[pl.ds(abase, rows_acc)],
                            idb[j].at[pl.ds(0, rows_acc)])

    def make_pn():
        pn_g = {}
        pn_o = {}

        def pn_fire(k):
            s = k % 2
            if k - 2 in pn_o:
                for dsc in pn_o.pop(k - 2):
                    dsc.wait()
            sl = pl.ds(k * CHUNK, CHUNK)
            pn_g[k] = [
                pltpu.async_copy(rep_hbm.at[pidv.at[sl]], pbuf[s], sg),
                pltpu.async_copy(rep_hbm.at[nidv.at[sl]], nbuf[s], sg),
            ]

        def pn_drain(k):
            s = k % 2
            for dsc in pn_g.pop(k):
                dsc.wait()
            base = pnbase + k * CHUNK
            pn_o[k] = [
                pltpu.async_copy(pbuf[s], pos_hbm.at[pl.ds(base, CHUNK)],
                                 sp),
                pltpu.async_copy(nbuf[s], neg_hbm.at[pl.ds(base, CHUNK)],
                                 sp),
            ]

        def pn_work(c):
            k = c // 2
            if k >= pn_chunks:
                return
            if c % 2 == 0:
                pn_fire(k)
            else:
                pn_drain(k)

        def pn_finish():
            for k in sorted(pn_g):
                pn_drain(k)
            for k in sorted(pn_o):
                for dsc in pn_o[k]:
                    dsc.wait()

        return pn_work, pn_finish

    @pl.when(is_a)
    def _():
        pn_work, pn_finish = make_pn()
        _acc_loop(tab, idb, 4, (ov0, ov1), acca_hbm, abase, nchunks, so,
                  pn_work, pn_finish)

    @pl.when(jnp.logical_not(is_a))
    def _():
        pn_work, pn_finish = make_pn()
        _acc_loop(tab, idb, 5, (ov0, ov1), acce_hbm, abase, nchunks, so,
                  pn_work, pn_finish)


def _gather(pa2, pe2, rep, aids, eids, pidx, nidx, batch):
    rows_acc = batch // 16
    rows_pn = batch // NW
    assert rows_acc % CHUNK == 0 and rows_pn % CHUNK == 0
    mesh = plsc.VectorSubcoreMesh(core_axis_name="c", subcore_axis_name="s",
                                  num_cores=NC, num_subcores=NS)
    out_t = jax.ShapeDtypeStruct((batch, D), jnp.float32)
    fn = pl.kernel(
        functools.partial(_gather_body, rows_acc=rows_acc, rows_pn=rows_pn),
        out_type=(out_t, out_t, out_t, out_t),
        mesh=mesh,
        compiler_params=pltpu.CompilerParams(needs_layout_passes=False),
        scratch_types=[
            pltpu.VMEM((VP // 2, D), jnp.int32),
            pltpu.VMEM((rows_acc + 16,), jnp.int32),
            pltpu.VMEM((rows_acc + 16,), jnp.int32),
            pltpu.VMEM((rows_acc + 16,), jnp.int32),
            pltpu.VMEM((rows_acc + 16,), jnp.int32),
            pltpu.VMEM((rows_acc + 16,), jnp.int32),
            pltpu.VMEM((rows_pn,), jnp.int32),
            pltpu.VMEM((rows_pn,), jnp.int32),
            pltpu.VMEM((CHUNK, D), jnp.float32),
            pltpu.VMEM((CHUNK, D), jnp.float32),
            pltpu.VMEM((CHUNK, D), jnp.float32),
            pltpu.VMEM((CHUNK, D), jnp.float32),
            pltpu.VMEM((CHUNK, D), jnp.float32),
            pltpu.VMEM((CHUNK, D), jnp.float32),
            pltpu.SemaphoreType.DMA, pltpu.SemaphoreType.DMA,
            pltpu.SemaphoreType.DMA,
        ],
    )
    return fn(pa2, pe2, rep, *aids, *eids, pidx, nidx)


def _head_body(acca_ref, acce_ref, pos_ref, neg_ref, cp_b1_ref, cp_g_ref,
               cp_bb_ref, cp_w2_ref, cp_b2_ref, ps_ref, ns_ref, *, blk):
    x = acca_ref[...] + acce_ref[...] + cp_b1_ref[...]
    mu = jnp.mean(x, axis=-1, keepdims=True)
    var = jnp.mean((x - mu) ** 2, axis=-1, keepdims=True)
    h = (x - mu) / jnp.sqrt(var + EPS) * cp_g_ref[...] + cp_bb_ref[...]
    h = jnp.maximum(h, 0.0)
    cv = jnp.dot(h, cp_w2_ref[...],
                 preferred_element_type=jnp.float32) + cp_b2_ref[...]
    ps = jnp.sum(cv * pos_ref[...], axis=-1)
    ns = jnp.sum(cv * neg_ref[...], axis=-1)
    ps_ref[...] = ps.reshape(blk // D, D)
    ns_ref[...] = ns.reshape(blk // D, D)


def _head(acca, acce, posv, negv, cp_b1_p, cp_g_p, cp_bb_p, cp_w2_p,
          cp_b2, batch):
    blk = 2048
    grid = (batch // blk,)
    bspec = pl.BlockSpec((blk, D), lambda i: (i, 0))
    wspec = pl.BlockSpec((1, D), lambda i: (0, 0))
    w2spec = pl.BlockSpec((D, D), lambda i: (0, 0))
    sspec = pl.BlockSpec((blk // D, D), lambda i: (i, 0))
    out = jax.ShapeDtypeStruct((batch // D, D), jnp.float32)
    return pl.pallas_call(
        functools.partial(_head_body, blk=blk),
        grid=grid,
        in_specs=[bspec, bspec, bspec, bspec, wspec, wspec, wspec, w2spec,
                  wspec],
        out_specs=(sspec, sspec),
        out_shape=(out, out),
    )(acca, acce, posv, negv, cp_b1_p.reshape(1, D), cp_g_p.reshape(1, D),
      cp_bb_p.reshape(1, D), cp_w2_p, cp_b2.reshape(1, D))


def kernel(ally_ids, enemy_ids, pos_hero_id, neg_hero_id, hero_emb,
           static_feats, sp_w, sp_b, sp_g, sp_bb, cp_w1, cp_b1, cp_g,
           cp_bb, cp_w2, cp_b2):
    batch = ally_ids.shape[0]
    v = hero_emb.shape[0]
    ally32 = ally_ids.astype(jnp.int32)
    enemy32 = enemy_ids.astype(jnp.int32)
    aids = [ally32[:, j] for j in range(4)]
    eids = [enemy32[:, j] for j in range(5)]
    pidx = pos_hero_id.astype(jnp.int32)
    nidx = neg_hero_id.astype(jnp.int32)
    perm = jnp.asarray(_PERM)
    cp_b1_p = cp_b1[perm]
    cp_g_p = cp_g[perm]
    cp_bb_p = cp_bb[perm]
    cp_w2_p = cp_w2[perm, :]

    pa, pe, rep = _tables(hero_emb, static_feats, sp_w, sp_b, sp_g, sp_bb,
                          cp_w1)
    pad = ((0, VP - v), (0, 0))
    pa2 = jnp.pad(pa, pad).reshape(VP // 2, D)
    pe2 = jnp.pad(pe, pad).reshape(VP // 2, D)
    acca, acce, posv, negv = _gather(pa2, pe2, rep, aids, eids, pidx, nidx,
                                     batch)
    ps, ns = _head(acca, acce, posv, negv, cp_b1_p, cp_g_p, cp_bb_p,
                   cp_w2_p, cp_b2, batch)
    return ps.reshape(batch), ns.reshape(batch)


# speedup vs baseline: 21.2277x; 1.2750x over previous
import functools

import jax
import jax.numpy as jnp
import numpy as np
from jax import lax
from jax.experimental import pallas as pl
from jax.experimental.pallas import tpu as pltpu
from jax.experimental.pallas import tpu_sc as plsc

EPS = 1e-5

NC = 2
NS = 16
NW = NC * NS
LANES = 16

D = 128
DW = D // 2
VP = 1002

_HI = -65536

_PERM = np.empty(D, np.int32)
for _p in range(D):
    _k, _l = divmod(_p, 16)
    if _k % 2 == 0:
        _PERM[_p] = 8 * _k + _l
    else:
        _PERM[_p] = 64 + 8 * (_k - 1) + _l


def _pack_rows(x):
    rb = x.astype(jnp.bfloat16).astype(jnp.float32)
    lo = lax.shift_right_logical(
        lax.bitcast_convert_type(rb[:, 0:DW], jnp.int32), 16)
    hi = lax.bitcast_convert_type(rb[:, DW:D], jnp.int32) & _HI
    return lo | hi


def _tables_body(hero_ref, stat_ref, sp_w_ref, sp_b_ref, sp_g_ref,
                 sp_bb_ref, cp_w1_ref, pa_ref, pe_ref, rep_ref):
    s = jnp.dot(stat_ref[...], sp_w_ref[...],
                preferred_element_type=jnp.float32) + sp_b_ref[...]
    mu = jnp.mean(s, axis=-1, keepdims=True)
    var = jnp.mean((s - mu) ** 2, axis=-1, keepdims=True)
    s = (s - mu) / jnp.sqrt(var + EPS) * sp_g_ref[...] + sp_bb_ref[...]
    rep = hero_ref[...] + 0.1 * jnp.tanh(s)
    pa = jnp.dot(rep, cp_w1_ref[0:D, :],
                 preferred_element_type=jnp.float32) * 0.25
    pe = jnp.dot(rep, cp_w1_ref[D:2 * D, :],
                 preferred_element_type=jnp.float32) * (0.8 / 5.0)
    pa_ref[...] = _pack_rows(pa)
    pe_ref[...] = _pack_rows(pe)
    rep_ref[...] = rep


def _tables(hero_emb, static_feats, sp_w, sp_b, sp_g, sp_bb, cp_w1):
    v = hero_emb.shape[0]
    packed = jax.ShapeDtypeStruct((v, DW), jnp.int32)
    repf = jax.ShapeDtypeStruct((v, D), jnp.float32)
    return pl.pallas_call(
        _tables_body,
        out_shape=(packed, packed, repf),
    )(hero_emb, static_feats, sp_w, sp_b.reshape(1, D), sp_g.reshape(1, D),
      sp_bb.reshape(1, D), cp_w1)


CHUNK = 64


def _acc_loop(tab, idb, nids, ov_pair, acc_hbm, abase, nchunks, sem_o,
              pn_work, pn_finish):
    iot = [lax.iota(jnp.int32, 16) + 16 * g for g in range(4)]
    od = {}
    for c in range(nchunks):
        ov = ov_pair[c % 2]
        if c - 2 in od:
            od.pop(c - 2).wait()

        @functools.partial(plsc.parallel_loop, 0, CHUNK, unroll=2)
        def _(i):
            acc_lo = [None] * 4
            acc_hi = [None] * 4
            for j in range(nids):
                idj = idb[j][pl.ds(c * CHUNK + i, 16)][0]
                row = jnp.full((16,), lax.shift_right_logical(idj, 1),
                               dtype=jnp.int32)
                rem = jnp.full((16,), (idj & 1) * DW, dtype=jnp.int32)
                for g in range(4):
                    w = plsc.load_gather(tab, [row, rem + iot[g]])
                    hf = plsc.bitcast(w & _HI, jnp.float32)
                    lf = plsc.bitcast(w << 16, jnp.float32)
                    acc_lo[g] = lf if acc_lo[g] is None else acc_lo[g] + lf
                    acc_hi[g] = hf if acc_hi[g] is None else acc_hi[g] + hf
            for g in range(4):
                ov[i, pl.ds(32 * g, LANES)] = acc_lo[g]
                ov[i, pl.ds(32 * g + LANES, LANES)] = acc_hi[g]
        od[c] = pltpu.async_copy(
            ov, acc_hbm.at[pl.ds(abase + c * CHUNK, CHUNK)], sem_o)
        pn_work(c)
    pn_finish()
    for c in sorted(od):
        od[c].wait()


def _gather_body(pa_hbm, pe_hbm, rep_hbm, a0h, a1h, a2h, a3h,
                 e0h, e1h, e2h, e3h, e4h, pidx_hbm, nidx_hbm,
                 acca_hbm, acce_hbm, pos_hbm, neg_hbm,
                 tab, i0, i1, i2, i3, i4, pidv, nidv,
                 pb0, pb1, nb0, nb1, ov0, ov1,
                 sg, so, sp, rows_acc, rows_pn):
    wid = lax.axis_index("s") * NC + lax.axis_index("c")
    is_a = wid < 16
    widr = lax.rem(wid, 16)
    abase = widr * rows_acc
    pnbase = wid * rows_pn
    nchunks = rows_acc // CHUNK
    pn_chunks = rows_pn // CHUNK
    idb = (i0, i1, i2, i3, i4)
    pbuf = (pb0, pb1)
    nbuf = (nb0, nb1)

    pltpu.sync_copy(pidx_hbm.at[pl.ds(pnbase, rows_pn)], pidv)
    pltpu.sync_copy(nidx_hbm.at[pl.ds(pnbase, rows_pn)], nidv)

    @pl.when(is_a)
    def _():
        pltpu.sync_copy(pa_hbm, tab)
        for j, h in enumerate((a0h, a1h, a2h, a3h)):
            pltpu.sync_copy(h.at[pl.ds(abase, rows_acc)],
                            idb[j].at[pl.ds(0, rows_acc)])

    @pl.when(jnp.logical_not(is_a))
    def _():
        pltpu.sync_copy(pe_hbm, tab)
        for j, h in enumerate((e0h, e1h, e2h, e3h, e4h)):
            pltpu.sync_copy(h.at[pl.ds(abase, rows_acc)],
                            idb[j].at[pl.ds(0, rows_acc)])

    def make_pn():
        pn_g = {}
        pn_o = {}

        def pn_fire(k):
            s = k % 2
            if k - 2 in pn_o:
                for dsc in pn_o.pop(k - 2):
                    dsc.wait()
            sl = pl.ds(k * CHUNK, CHUNK)
            pn_g[k] = [
                pltpu.async_copy(rep_hbm.at[pidv.at[sl]], pbuf[s], sg),
                pltpu.async_copy(rep_hbm.at[nidv.at[sl]], nbuf[s], sg),
            ]

        def pn_drain(k):
            s = k % 2
            for dsc in pn_g.pop(k):
                dsc.wait()
            base = pnbase + k * CHUNK
            pn_o[k] = [
                pltpu.async_copy(pbuf[s], pos_hbm.at[pl.ds(base, CHUNK)],
                                 sp),
                pltpu.async_copy(nbuf[s], neg_hbm.at[pl.ds(base, CHUNK)],
                                 sp),
            ]

        def pn_work(c):
            k = c // 2
            if k >= pn_chunks:
                return
            if c % 2 == 0:
                pn_fire(k)
            else:
                pn_drain(k)

        def pn_finish():
            for k in sorted(pn_g):
                pn_drain(k)
            for k in sorted(pn_o):
                for dsc in pn_o[k]:
                    dsc.wait()

        return pn_work, pn_finish

    @pl.when(is_a)
    def _():
        pn_work, pn_finish = make_pn()
        _acc_loop(tab, idb, 4, (ov0, ov1), acca_hbm, abase, nchunks, so,
                  pn_work, pn_finish)

    @pl.when(jnp.logical_not(is_a))
    def _():
        pn_work, pn_finish = make_pn()
        _acc_loop(tab, idb, 5, (ov0, ov1), acce_hbm, abase, nchunks, so,
                  pn_work, pn_finish)


def _gather(pa2, pe2, rep, aids, eids, pidx, nidx, batch):
    rows_acc = batch // 16
    rows_pn = batch // NW
    assert rows_acc % CHUNK == 0 and rows_pn % CHUNK == 0
    mesh = plsc.VectorSubcoreMesh(core_axis_name="c", subcore_axis_name="s",
                                  num_cores=NC, num_subcores=NS)
    out_t = jax.ShapeDtypeStruct((batch, D), jnp.float32)
    fn = pl.kernel(
        functools.partial(_gather_body, rows_acc=rows_acc, rows_pn=rows_pn),
        out_type=(out_t, out_t, out_t, out_t),
        mesh=mesh,
        compiler_params=pltpu.CompilerParams(needs_layout_passes=False),
        scratch_types=[
            pltpu.VMEM((VP // 2, D), jnp.int32),
            pltpu.VMEM((rows_acc + 16,), jnp.int32),
            pltpu.VMEM((rows_acc + 16,), jnp.int32),
            pltpu.VMEM((rows_acc + 16,), jnp.int32),
            pltpu.VMEM((rows_acc + 16,), jnp.int32),
            pltpu.VMEM((rows_acc + 16,), jnp.int32),
            pltpu.VMEM((rows_pn,), jnp.int32),
            pltpu.VMEM((rows_pn,), jnp.int32),
            pltpu.VMEM((CHUNK, D), jnp.float32),
            pltpu.VMEM((CHUNK, D), jnp.float32),
            pltpu.VMEM((CHUNK, D), jnp.float32),
            pltpu.VMEM((CHUNK, D), jnp.float32),
            pltpu.VMEM((CHUNK, D), jnp.float32),
            pltpu.VMEM((CHUNK, D), jnp.float32),
            pltpu.SemaphoreType.DMA, pltpu.SemaphoreType.DMA,
            pltpu.SemaphoreType.DMA,
        ],
    )
    return fn(pa2, pe2, rep, *aids, *eids, pidx, nidx)


def _head_body(acca_ref, acce_ref, pos_ref, neg_ref, cp_b1_ref, cp_g_ref,
               cp_bb_ref, cp_w2_ref, cp_b2_ref, ps_ref, ns_ref, *, blk):
    x = acca_ref[...] + acce_ref[...] + cp_b1_ref[...]
    mu = jnp.mean(x, axis=-1, keepdims=True)
    var = jnp.mean((x - mu) ** 2, axis=-1, keepdims=True)
    h = (x - mu) / jnp.sqrt(var + EPS) * cp_g_ref[...] + cp_bb_ref[...]
    h = jnp.maximum(h, 0.0)
    cv = jnp.dot(h, cp_w2_ref[...],
                 preferred_element_type=jnp.float32) + cp_b2_ref[...]
    ps = jnp.sum(cv * pos_ref[...], axis=-1)
    ns = jnp.sum(cv * neg_ref[...], axis=-1)
    ps_ref[...] = ps.reshape(blk // D, D)
    ns_ref[...] = ns.reshape(blk // D, D)


def _head(acca, acce, posv, negv, cp_b1_p, cp_g_p, cp_bb_p, cp_w2_p,
          cp_b2, batch):
    blk = 2048
    grid = (batch // blk,)
    bspec = pl.BlockSpec((blk, D), lambda i: (i, 0))
    wspec = pl.BlockSpec((1, D), lambda i: (0, 0))
    w2spec = pl.BlockSpec((D, D), lambda i: (0, 0))
    sspec = pl.BlockSpec((blk // D, D), lambda i: (i, 0))
    out = jax.ShapeDtypeStruct((batch // D, D), jnp.float32)
    return pl.pallas_call(
        functools.partial(_head_body, blk=blk),
        grid=grid,
        in_specs=[bspec, bspec, bspec, bspec, wspec, wspec, wspec, w2spec,
                  wspec],
        out_specs=(sspec, sspec),
        out_shape=(out, out),
    )(acca, acce, posv, negv, cp_b1_p.reshape(1, D), cp_g_p.reshape(1, D),
      cp_bb_p.reshape(1, D), cp_w2_p, cp_b2.reshape(1, D))


def kernel(ally_ids, enemy_ids, pos_hero_id, neg_hero_id, hero_emb,
           static_feats, sp_w, sp_b, sp_g, sp_bb, cp_w1, cp_b1, cp_g,
           cp_bb, cp_w2, cp_b2):
    batch = ally_ids.shape[0]
    v = hero_emb.shape[0]
    ally32 = ally_ids.astype(jnp.int32)
    enemy32 = enemy_ids.astype(jnp.int32)
    aids = [ally32[:, j] for j in range(4)]
    eids = [enemy32[:, j] for j in range(5)]
    pidx = pos_hero_id.astype(jnp.int32)
    nidx = neg_hero_id.astype(jnp.int32)
    perm = jnp.asarray(_PERM)
    cp_b1_p = cp_b1[perm]
    cp_g_p = cp_g[perm]
    cp_bb_p = cp_bb[perm]
    cp_w2_p = cp_w2[perm, :]

    pa, pe, rep = _tables(hero_emb, static_feats, sp_w, sp_b, sp_g, sp_bb,
                          cp_w1)
    pad = ((0, VP - v), (0, 0))
    pa2 = jnp.pad(pa, pad).reshape(VP // 2, D)
    pe2 = jnp.pad(pe, pad).reshape(VP // 2, D)
    acca, acce, posv, negv = _gather(pa2, pe2, rep, aids, eids, pidx, nidx,
                                     batch)
    ps, ns = _head(acca, acce, posv, negv, cp_b1_p, cp_g_p, cp_bb_p,
                   cp_w2_p, cp_b2, batch)
    return ps.reshape(batch), ns.reshape(batch)
